# Initial kernel scaffold; baseline (speedup 1.0000x reference)
import jax
import jax.numpy as jnp
from jax.experimental import pallas as pl


def kernel(features, weight_theta, weight_bias, bias, neighborhood, positions):
    raise NotImplementedError("write your pallas kernel here")


# speedup vs baseline: 6.0262x; 6.0262x over previous
import functools

import jax
import jax.numpy as jnp
from jax import lax
from jax.experimental import pallas as pl
from jax.experimental.pallas import tpu as pltpu
from jax.experimental.pallas import tpu_sc as plsc

NT = 16
NC = 2
SEG = 128
BN = 1024


def _prep_body(f_ref, w_ref, pos_ref, h_ref):
    p = lax.dot_general(f_ref[...], w_ref[...], (((0,), (0,)), ((), ())),
                        preferred_element_type=jnp.float32)
    g = p[:, 384:512]
    for d in range(3):
        g = g - pos_ref[d, :][:, None] * p[:, d * 128:(d + 1) * 128]
    h_ref[0] = p[:, 0:128]
    h_ref[1] = p[:, 128:256]
    h_ref[2] = p[:, 256:384]
    h_ref[3] = g


def _comb_body(s_ref, pos_ref, b_ref, o_ref):
    acc = s_ref[3] + b_ref[0, :][None, :]
    for d in range(3):
        acc = acc + pos_ref[d, :][:, None] * s_ref[d]
    o_ref[...] = acc.T


def _make_sc_scatter(npad, npt, k):
    nseg = npt // SEG
    nrow = k * nseg
    mesh = plsc.VectorSubcoreMesh(core_axis_name="c", subcore_axis_name="s")

    @functools.partial(
        pl.kernel,
        out_type=jax.ShapeDtypeStruct((4, npad, 128), jnp.float32),
        mesh=mesh,
        scratch_types=[
            pltpu.VMEM((SEG, 128), jnp.float32),
            pltpu.VMEM((nrow, 128), jnp.int32),
            pltpu.VMEM_SHARED((npad, 128), jnp.float32),
        ],
    )
    def sc_scatter(h4_hbm, idx_hbm, z_hbm, s4_hbm, h_v, idx_v, shared):
        c = lax.axis_index("c")
        s = lax.axis_index("s")
        base = s * npt
        pltpu.sync_copy(idx_hbm.at[s], idx_v)
        for cc in range(2):
            chunk = c * 2 + cc
            pltpu.sync_copy(z_hbm, shared.at[pl.ds(base, npt)])
            plsc.subcore_barrier()

            for seg in range(nseg):
                pltpu.sync_copy(h4_hbm.at[chunk, pl.ds(base + seg * SEG, SEG)], h_v)

                def kbody(kk, carry):
                    pltpu.sync_copy(h_v,
                                    shared.at[idx_v.at[seg * k + kk]],
                                    add=True)
                    return carry

                lax.fori_loop(0, k, kbody, 0)
            plsc.subcore_barrier()
            pltpu.sync_copy(shared.at[pl.ds(base, npt)],
                            s4_hbm.at[chunk, pl.ds(base, npt)])

    return sc_scatter


def kernel(features, weight_theta, weight_bias, bias, neighborhood, positions):
    b, din, n = features.shape
    k = neighborhood.shape[1]
    dout = weight_theta.shape[-1]
    npt = ((n + NT * SEG - 1) // (NT * SEG)) * SEG
    npad = npt * NT
    nseg = npt // SEG

    f_pad = jnp.pad(features[0], ((0, 0), (0, npad - n)))
    pos8 = jnp.pad(positions[0], ((0, 5), (0, npad - n)))
    wcat = jnp.concatenate(
        [weight_theta[0], weight_theta[1], weight_theta[2], weight_bias], axis=1)
    bias_pad = jnp.pad(bias[None, :], ((0, 7), (0, 0)))
    nb_pad = jnp.pad(neighborhood[0], ((0, 0), (0, npad - n)))
    idx = nb_pad.reshape(k, NT, nseg, SEG).transpose(1, 2, 0, 3).reshape(NT, k * nseg, SEG)
    z = jnp.zeros((npt, 128), jnp.float32)

    grid = (npad // BN,)
    h4 = pl.pallas_call(
        _prep_body,
        grid=grid,
        in_specs=[
            pl.BlockSpec((din, BN), lambda i: (0, i)),
            pl.BlockSpec((din, 4 * dout), lambda i: (0, 0)),
            pl.BlockSpec((8, BN), lambda i: (0, i)),
        ],
        out_specs=pl.BlockSpec((4, BN, dout), lambda i: (0, i, 0)),
        out_shape=jax.ShapeDtypeStruct((4, npad, dout), jnp.float32),
    )(f_pad, wcat, pos8)

    s4 = _make_sc_scatter(npad, npt, k)(h4, idx, z)

    o_t = pl.pallas_call(
        _comb_body,
        grid=grid,
        in_specs=[
            pl.BlockSpec((4, BN, dout), lambda i: (0, i, 0)),
            pl.BlockSpec((8, BN), lambda i: (0, i)),
            pl.BlockSpec((8, dout), lambda i: (0, 0)),
        ],
        out_specs=pl.BlockSpec((dout, BN), lambda i: (0, i)),
        out_shape=jax.ShapeDtypeStruct((dout, npad), jnp.float32),
    )(s4, pos8, bias_pad)

    return o_t[None, :, :n]
